# Initial kernel scaffold; baseline (speedup 1.0000x reference)
import jax
import jax.numpy as jnp
from jax.experimental import pallas as pl


def kernel(x, edge_index, Wl1, bl1, Wr1, Wl2, bl2, Wr2, Wl3, bl3, Wr3, fc1_w, fc1_b, fc2_w, fc2_b):
    raise NotImplementedError("write your pallas kernel here")


# speedup vs baseline: 11.4472x; 11.4472x over previous
import functools

import jax
import jax.numpy as jnp
from jax import lax
from jax.experimental import pallas as pl
from jax.experimental.pallas import tpu as pltpu
from jax.experimental.pallas import tpu_sc as plsc

N_NODES = 100000
N_EDGES = 1600000

NC = 2
NS = 16
NW = NC * NS
CHUNK = 1024
STREAMS = 8
CHUNKS_PER_TILE = 49
EDGES_PER_TILE = CHUNKS_PER_TILE * CHUNK
E_PAD = NW * EDGES_PER_TILE
IDX_ROWS = E_PAD // 128
ROWS_PER_TILE_IDX = EDGES_PER_TILE // 128
N_ACC = 102400
NODES_PER_TILE = N_ACC // NS

D = 16

@functools.cache
def _build_sc_segment_sum():
    mesh = plsc.VectorSubcoreMesh(
        core_axis_name="c", subcore_axis_name="s",
        num_cores=NC, num_subcores=NS)
    return functools.partial(
        pl.kernel,
        out_type=jax.ShapeDtypeStruct((NC, N_ACC, D), jnp.float32),
        mesh=mesh,
        scratch_types=[
            pltpu.VMEM((STREAMS, 128), jnp.int32),
            pltpu.VMEM((STREAMS, 128), jnp.int32),
            pltpu.VMEM((CHUNK, D), jnp.float32),
            pltpu.VMEM_SHARED((N_ACC, D), jnp.float32),
            pltpu.SemaphoreType.DMA,
        ],
        compiler_params=pltpu.CompilerParams(use_tc_tiling_on_sc=False),
    )(_sc_segment_sum_body)


def _sc_segment_sum(g_pad, src2d, dst2d):
    return _build_sc_segment_sum()(g_pad, src2d, dst2d)


def _sc_segment_sum_body(g_hbm, src_hbm, dst_hbm, out_hbm,
                         idx_v, dst_v, rows_v, acc, sem):
    cid = lax.axis_index("c")
    sid = lax.axis_index("s")
    wid = cid * NS + sid

    def zero_body(i, carry):
        rows_v[i] = jnp.zeros((D,), jnp.float32)
        return carry
    lax.fori_loop(0, CHUNK, zero_body, 0)
    base = sid * NODES_PER_TILE
    for t in range(NODES_PER_TILE // CHUNK):
        pltpu.sync_copy(rows_v, acc.at[pl.ds(base + t * CHUNK, CHUNK)])
    rem = NODES_PER_TILE % CHUNK
    if rem:
        pltpu.sync_copy(rows_v.at[pl.ds(0, rem)],
                        acc.at[pl.ds(base + NODES_PER_TILE - rem, rem)])
    plsc.subcore_barrier()

    row_base = wid * ROWS_PER_TILE_IDX

    def chunk_body(k, carry):
        r0 = row_base + k * STREAMS
        pltpu.sync_copy(src_hbm.at[pl.ds(r0, STREAMS)], idx_v)
        descs = [
            pltpu.async_copy(g_hbm.at[idx_v.at[j]],
                             rows_v.at[pl.ds(j * 128, 128)], sem)
            for j in range(STREAMS)
        ]
        for d in descs:
            d.wait()
        pltpu.sync_copy(dst_hbm.at[pl.ds(r0, STREAMS)], dst_v)
        for j in range(STREAMS):
            pltpu.sync_copy(rows_v.at[pl.ds(j * 128, 128)],
                            acc.at[dst_v.at[j]], add=True)
        return carry
    lax.fori_loop(0, CHUNKS_PER_TILE, chunk_body, 0)
    plsc.subcore_barrier()

    pltpu.sync_copy(acc.at[pl.ds(sid * NODES_PER_TILE, NODES_PER_TILE)],
                    out_hbm.at[cid, pl.ds(sid * NODES_PER_TILE, NODES_PER_TILE)])


BN = 4000


def _dot(a, b):
    return jax.lax.dot_general(a, b, (((1,), (0,)), ((), ())),
                               preferred_element_type=jnp.float32,
                               precision=jax.lax.Precision.HIGHEST)


def _dense1_body(p_ref, x_ref, wl1t, bl1, wr1t, wl2t, wr2t,
                 g2_ref, r2_ref, dinv_ref):
    p = p_ref[...]
    agg = p[0] + p[1]
    dinv = 1.0 / jnp.maximum(agg[:, 8:9], 1.0)
    mean = agg[:, 0:8] * dinv
    h1 = jnp.maximum(_dot(mean, wl1t[...]) + bl1[...] + _dot(x_ref[...], wr1t[...]), 0.0)
    g2_ref[...] = _dot(h1, wl2t[...])
    r2_ref[...] = _dot(h1, wr2t[...])
    dinv_ref[...] = jnp.broadcast_to(dinv, (dinv.shape[0], 8))


def _dense2_body(p2a_ref, p2b_ref, r2_ref, dinv_ref, bl2, wl3t, wr3t,
                 g3_ref, r3_ref):
    p2a = p2a_ref[...]
    p2b = p2b_ref[...]
    agg2 = jnp.concatenate([p2a[0] + p2a[1], p2b[0] + p2b[1]], axis=1)
    dinv = dinv_ref[...][:, 0:1]
    h2 = jnp.maximum(agg2 * dinv + bl2[...] + r2_ref[...], 0.0)
    g3_ref[...] = _dot(h2, wl3t[...])
    r3_ref[...] = _dot(h2, wr3t[...])


def _dense3_body(p3_ref, r3_ref, dinv_ref, bl3, fc1t, fc1b, fc2t, fc2b,
                 out_ref):
    p3 = p3_ref[...]
    agg3 = p3[0] + p3[1]
    dinv = dinv_ref[...][:, 0:1]
    h3 = jnp.maximum(agg3 * dinv + bl3[...] + r3_ref[...], 0.0)
    f1 = jnp.maximum(_dot(h3, fc1t[...]) + fc1b[...], 0.0)
    lg = _dot(f1, fc2t[...]) + fc2b[...]
    m = jnp.max(lg, axis=1, keepdims=True)
    out_ref[...] = lg - (m + jnp.log(jnp.sum(jnp.exp(lg - m), axis=1,
                                             keepdims=True)))


def _row_spec(width):
    return pl.BlockSpec((BN, width), lambda i: (i, 0))


def _p_spec():
    return pl.BlockSpec((2, BN, D), lambda i: (0, i, 0))


def _w_spec(shape):
    return pl.BlockSpec(shape, lambda i: tuple(0 for _ in shape))


def _pad_rows(a):
    return jnp.concatenate([a, jnp.zeros((1, a.shape[1]), a.dtype)], axis=0)


def kernel(x, edge_index, Wl1, bl1, Wr1, Wl2, bl2, Wr2, Wl3, bl3, Wr3,
           fc1_w, fc1_b, fc2_w, fc2_b):
    n = x.shape[0]
    e = edge_index.shape[1]
    assert n == N_NODES and e == N_EDGES

    pad = E_PAD - e
    src2d = jnp.concatenate(
        [edge_index[0], jnp.full((pad,), n, jnp.int32)]).reshape(IDX_ROWS, 128)
    dst2d = jnp.concatenate(
        [edge_index[1], jnp.zeros((pad,), jnp.int32)]).reshape(IDX_ROWS, 128)

    g1 = jnp.concatenate(
        [x, jnp.ones((n, 1), x.dtype), jnp.zeros((n, 7), x.dtype)], axis=1)
    p1 = _sc_segment_sum(_pad_rows(g1), src2d, dst2d)

    grid = (N_NODES // BN,)
    g2, r2, dinv = pl.pallas_call(
        _dense1_body,
        grid=grid,
        in_specs=[_p_spec(), _row_spec(8), _w_spec((8, 64)), _w_spec((1, 64)),
                  _w_spec((8, 64)), _w_spec((64, 32)), _w_spec((64, 32))],
        out_specs=[_row_spec(32), _row_spec(32), _row_spec(8)],
        out_shape=[jax.ShapeDtypeStruct((n, 32), jnp.float32),
                   jax.ShapeDtypeStruct((n, 32), jnp.float32),
                   jax.ShapeDtypeStruct((n, 8), jnp.float32)],
    )(p1, x, Wl1.T, bl1.reshape(1, 64), Wr1.T, Wl2.T, Wr2.T)

    p2a = _sc_segment_sum(_pad_rows(g2[:, 0:16]), src2d, dst2d)
    p2b = _sc_segment_sum(_pad_rows(g2[:, 16:32]), src2d, dst2d)

    g3, r3 = pl.pallas_call(
        _dense2_body,
        grid=grid,
        in_specs=[_p_spec(), _p_spec(), _row_spec(32), _row_spec(8),
                  _w_spec((1, 32)), _w_spec((32, 16)), _w_spec((32, 16))],
        out_specs=[_row_spec(16), _row_spec(16)],
        out_shape=[jax.ShapeDtypeStruct((n, 16), jnp.float32),
                   jax.ShapeDtypeStruct((n, 16), jnp.float32)],
    )(p2a, p2b, r2, dinv, bl2.reshape(1, 32), Wl3.T, Wr3.T)

    p3 = _sc_segment_sum(_pad_rows(g3), src2d, dst2d)

    out = pl.pallas_call(
        _dense3_body,
        grid=grid,
        in_specs=[_p_spec(), _row_spec(16), _row_spec(8), _w_spec((1, 16)),
                  _w_spec((16, 8)), _w_spec((1, 8)), _w_spec((8, 2)),
                  _w_spec((1, 2))],
        out_specs=[_row_spec(2)],
        out_shape=[jax.ShapeDtypeStruct((n, 2), jnp.float32)],
    )(p3, r3, dinv, bl3.reshape(1, 16), fc1_w.T, fc1_b.reshape(1, 8),
      fc2_w.T, fc2_b.reshape(1, 2))[0]

    return out


# speedup vs baseline: 14.6508x; 1.2799x over previous
import functools

import jax
import jax.numpy as jnp
from jax import lax
from jax.experimental import pallas as pl
from jax.experimental.pallas import tpu as pltpu
from jax.experimental.pallas import tpu_sc as plsc

N_NODES = 100000
N_EDGES = 1600000

NC = 2
NS = 16
NW = NC * NS
CHUNK = 1024
STREAMS = 8
CHUNKS_PER_TILE = 49
EDGES_PER_TILE = CHUNKS_PER_TILE * CHUNK
E_PAD = NW * EDGES_PER_TILE
IDX_ROWS = E_PAD // 128
ROWS_PER_TILE_IDX = EDGES_PER_TILE // 128
N_PAD = 102400
NODES_PER_TILE = N_PAD // NS
D = 16


@functools.cache
def _build_sc_segment_sum():
    mesh = plsc.VectorSubcoreMesh(
        core_axis_name="c", subcore_axis_name="s",
        num_cores=NC, num_subcores=NS)
    return functools.partial(
        pl.kernel,
        out_type=jax.ShapeDtypeStruct((NC, N_PAD, D), jnp.float32),
        mesh=mesh,
        scratch_types=[
            pltpu.VMEM((STREAMS, 128), jnp.int32),
            pltpu.VMEM((STREAMS, 128), jnp.int32),
            pltpu.VMEM((CHUNK, D), jnp.float32),
            pltpu.VMEM_SHARED((N_PAD, D), jnp.float32),
            pltpu.SemaphoreType.DMA,
        ],
        compiler_params=pltpu.CompilerParams(use_tc_tiling_on_sc=False),
    )(_sc_segment_sum_body)


def _sc_segment_sum(g_pad, edges2d):
    return _build_sc_segment_sum()(g_pad, edges2d)


def _sc_segment_sum_body(g_hbm, edges_hbm, out_hbm,
                         idx_v, dst_v, rows_v, acc, sem):
    cid = lax.axis_index("c")
    sid = lax.axis_index("s")
    wid = cid * NS + sid

    def zero_body(i, carry):
        rows_v[i] = jnp.zeros((D,), jnp.float32)
        return carry
    lax.fori_loop(0, CHUNK, zero_body, 0)
    base = sid * NODES_PER_TILE
    for t in range(NODES_PER_TILE // CHUNK):
        pltpu.sync_copy(rows_v, acc.at[pl.ds(base + t * CHUNK, CHUNK)])
    rem = NODES_PER_TILE % CHUNK
    if rem:
        pltpu.sync_copy(rows_v.at[pl.ds(0, rem)],
                        acc.at[pl.ds(base + NODES_PER_TILE - rem, rem)])
    plsc.subcore_barrier()

    row_base = wid * ROWS_PER_TILE_IDX

    def chunk_body(k, carry):
        r0 = row_base + k * STREAMS
        pltpu.sync_copy(edges_hbm.at[0, pl.ds(r0, STREAMS)], idx_v)
        descs = [
            pltpu.async_copy(g_hbm.at[idx_v.at[j]],
                             rows_v.at[pl.ds(j * 128, 128)], sem)
            for j in range(STREAMS)
        ]
        for d in descs:
            d.wait()
        pltpu.sync_copy(edges_hbm.at[1, pl.ds(r0, STREAMS)], dst_v)
        for j in range(STREAMS):
            pltpu.sync_copy(rows_v.at[pl.ds(j * 128, 128)],
                            acc.at[dst_v.at[j]], add=True)
        return carry
    lax.fori_loop(0, CHUNKS_PER_TILE, chunk_body, 0)
    plsc.subcore_barrier()

    pltpu.sync_copy(acc.at[pl.ds(sid * NODES_PER_TILE, NODES_PER_TILE)],
                    out_hbm.at[cid, pl.ds(sid * NODES_PER_TILE, NODES_PER_TILE)])


BN = 4096


def _dot(a, b):
    return jax.lax.dot_general(a, b, (((1,), (0,)), ((), ())),
                               preferred_element_type=jnp.float32)


def _row_mask(i, bn):
    row = i * bn + lax.broadcasted_iota(jnp.int32, (bn, 1), 0)
    return jnp.where(row < N_NODES, 1.0, 0.0).astype(jnp.float32)


def _dense1_body(p_ref, x_ref, wl1t, bl1, wr1t, wl2t, wr2t,
                 g2a_ref, g2b_ref, r2_ref, dinv_ref):
    i = pl.program_id(0)
    p = p_ref[...]
    agg = p[0] + p[1]
    dinv = 1.0 / jnp.maximum(agg[:, 8:9], 1.0)
    mean = agg[:, 0:8] * dinv
    h1 = jnp.maximum(_dot(mean, wl1t[...]) + bl1[...] + _dot(x_ref[...], wr1t[...]), 0.0)
    g2 = _dot(h1, wl2t[...]) * _row_mask(i, h1.shape[0])
    g2a_ref[...] = g2[:, 0:16]
    g2b_ref[...] = g2[:, 16:32]
    r2_ref[...] = _dot(h1, wr2t[...])
    dinv_ref[...] = jnp.broadcast_to(dinv, (dinv.shape[0], 8))


def _dense2_body(p2a_ref, p2b_ref, r2_ref, dinv_ref, bl2, wl3t, wr3t,
                 g3_ref, r3_ref):
    i = pl.program_id(0)
    p2a = p2a_ref[...]
    p2b = p2b_ref[...]
    agg2 = jnp.concatenate([p2a[0] + p2a[1], p2b[0] + p2b[1]], axis=1)
    dinv = dinv_ref[...][:, 0:1]
    h2 = jnp.maximum(agg2 * dinv + bl2[...] + r2_ref[...], 0.0)
    g3_ref[...] = _dot(h2, wl3t[...]) * _row_mask(i, h2.shape[0])
    r3_ref[...] = _dot(h2, wr3t[...])


def _dense3_body(p3_ref, r3_ref, dinv_ref, bl3, fc1t, fc1b, fc2t, fc2b,
                 out_ref):
    p3 = p3_ref[...]
    agg3 = p3[0] + p3[1]
    dinv = dinv_ref[...][:, 0:1]
    h3 = jnp.maximum(agg3 * dinv + bl3[...] + r3_ref[...], 0.0)
    f1 = jnp.maximum(_dot(h3, fc1t[...]) + fc1b[...], 0.0)
    lg = _dot(f1, fc2t[...]) + fc2b[...]
    m = jnp.max(lg, axis=1, keepdims=True)
    out_ref[...] = lg - (m + jnp.log(jnp.sum(jnp.exp(lg - m), axis=1,
                                             keepdims=True)))


def _row_spec(width):
    return pl.BlockSpec((BN, width), lambda i: (i, 0))


def _p_spec():
    return pl.BlockSpec((2, BN, D), lambda i: (0, i, 0))


def _w_spec(shape):
    return pl.BlockSpec(shape, lambda i: tuple(0 for _ in shape))


def kernel(x, edge_index, Wl1, bl1, Wr1, Wl2, bl2, Wr2, Wl3, bl3, Wr3,
           fc1_w, fc1_b, fc2_w, fc2_b):
    n = x.shape[0]
    e = edge_index.shape[1]
    assert n == N_NODES and e == N_EDGES

    edges2d = jnp.pad(edge_index, ((0, 0), (0, E_PAD - e)),
                      constant_values=n).reshape(2, IDX_ROWS, 128)

    g1 = jnp.pad(
        jnp.concatenate([x, jnp.ones((n, 1), x.dtype)], axis=1),
        ((0, N_PAD - n), (0, D - 9)))
    p1 = _sc_segment_sum(g1, edges2d)

    grid = (N_PAD // BN,)
    g2a, g2b, r2, dinv = pl.pallas_call(
        _dense1_body,
        grid=grid,
        in_specs=[_p_spec(), _row_spec(8), _w_spec((8, 64)), _w_spec((1, 64)),
                  _w_spec((8, 64)), _w_spec((64, 32)), _w_spec((64, 32))],
        out_specs=[_row_spec(16), _row_spec(16), _row_spec(32), _row_spec(8)],
        out_shape=[jax.ShapeDtypeStruct((N_PAD, 16), jnp.float32),
                   jax.ShapeDtypeStruct((N_PAD, 16), jnp.float32),
                   jax.ShapeDtypeStruct((N_PAD, 32), jnp.float32),
                   jax.ShapeDtypeStruct((N_PAD, 8), jnp.float32)],
    )(p1, x, Wl1.T, bl1.reshape(1, 64), Wr1.T, Wl2.T, Wr2.T)

    p2a = _sc_segment_sum(g2a, edges2d)
    p2b = _sc_segment_sum(g2b, edges2d)

    g3, r3 = pl.pallas_call(
        _dense2_body,
        grid=grid,
        in_specs=[_p_spec(), _p_spec(), _row_spec(32), _row_spec(8),
                  _w_spec((1, 32)), _w_spec((32, 16)), _w_spec((32, 16))],
        out_specs=[_row_spec(16), _row_spec(16)],
        out_shape=[jax.ShapeDtypeStruct((N_PAD, 16), jnp.float32),
                   jax.ShapeDtypeStruct((N_PAD, 16), jnp.float32)],
    )(p2a, p2b, r2, dinv, bl2.reshape(1, 32), Wl3.T, Wr3.T)

    p3 = _sc_segment_sum(g3, edges2d)

    out = pl.pallas_call(
        _dense3_body,
        grid=grid,
        in_specs=[_p_spec(), _row_spec(16), _row_spec(8), _w_spec((1, 16)),
                  _w_spec((16, 8)), _w_spec((1, 8)), _w_spec((8, 2)),
                  _w_spec((1, 2))],
        out_specs=[_row_spec(2)],
        out_shape=[jax.ShapeDtypeStruct((N_NODES, 2), jnp.float32)],
    )(p3, r3, dinv, bl3.reshape(1, 16), fc1_w.T, fc1_b.reshape(1, 8),
      fc2_w.T, fc2_b.reshape(1, 2))[0]

    return out


# speedup vs baseline: 20.0520x; 1.3687x over previous
import functools

import jax
import jax.numpy as jnp
import numpy as np
from jax import lax
from jax.experimental import pallas as pl
from jax.experimental.pallas import tpu as pltpu
from jax.experimental.pallas import tpu_sc as plsc

N_NODES = 100000
N_EDGES = 1600000

NC = 2
NS = 16
NW = NC * NS
CHUNK = 1024
STREAMS = 8
CHUNKS_PER_TILE = 49
EDGES_PER_TILE = CHUNKS_PER_TILE * CHUNK
E_PAD = NW * EDGES_PER_TILE
IDX_ROWS = E_PAD // 128
ROWS_PER_TILE_IDX = EDGES_PER_TILE // 128
N_PAD = 102400
NODES_PER_TILE = N_PAD // NS
D = 16
NP128 = N_PAD * D // 128
NV128 = N_NODES * D // 128


@functools.cache
def _build_sc_segment_sum():
    mesh = plsc.VectorSubcoreMesh(
        core_axis_name="c", subcore_axis_name="s",
        num_cores=NC, num_subcores=NS)
    return functools.partial(
        pl.kernel,
        out_type=jax.ShapeDtypeStruct((NC, N_PAD, D), jnp.float32),
        mesh=mesh,
        scratch_types=[
            pltpu.VMEM((STREAMS, 128), jnp.int32),
            pltpu.VMEM((STREAMS, 128), jnp.int32),
            pltpu.VMEM((CHUNK, D), jnp.float32),
            pltpu.VMEM_SHARED((N_PAD, D), jnp.float32),
            pltpu.SemaphoreType.DMA,
        ],
        compiler_params=pltpu.CompilerParams(use_tc_tiling_on_sc=False),
    )(_sc_segment_sum_body)


def _sc_segment_sum(g128, edges2d):
    p = _build_sc_segment_sum()(g128.reshape(N_PAD, D), edges2d)
    return p.reshape(NC, NP128, 128)


def _sc_segment_sum_body(g_hbm, edges_hbm, out_hbm,
                         idx_v, dst_v, rows_v, acc, sem):
    cid = lax.axis_index("c")
    sid = lax.axis_index("s")
    wid = cid * NS + sid

    def zero_body(i, carry):
        rows_v[i] = jnp.zeros((D,), jnp.float32)
        return carry
    lax.fori_loop(0, CHUNK, zero_body, 0)
    base = sid * NODES_PER_TILE
    for t in range(NODES_PER_TILE // CHUNK):
        pltpu.sync_copy(rows_v, acc.at[pl.ds(base + t * CHUNK, CHUNK)])
    rem = NODES_PER_TILE % CHUNK
    if rem:
        pltpu.sync_copy(rows_v.at[pl.ds(0, rem)],
                        acc.at[pl.ds(base + NODES_PER_TILE - rem, rem)])
    plsc.subcore_barrier()

    row_base = wid * ROWS_PER_TILE_IDX

    def chunk_body(k, carry):
        r0 = row_base + k * STREAMS
        pltpu.sync_copy(edges_hbm.at[0, pl.ds(r0, STREAMS)], idx_v)
        descs = [
            pltpu.async_copy(g_hbm.at[idx_v.at[j]],
                             rows_v.at[pl.ds(j * 128, 128)], sem)
            for j in range(STREAMS)
        ]
        for d in descs:
            d.wait()
        pltpu.sync_copy(edges_hbm.at[1, pl.ds(r0, STREAMS)], dst_v)
        for j in range(STREAMS):
            pltpu.sync_copy(rows_v.at[pl.ds(j * 128, 128)],
                            acc.at[dst_v.at[j]], add=True)
        return carry
    lax.fori_loop(0, CHUNKS_PER_TILE, chunk_body, 0)
    plsc.subcore_barrier()

    pltpu.sync_copy(acc.at[pl.ds(sid * NODES_PER_TILE, NODES_PER_TILE)],
                    out_hbm.at[cid, pl.ds(sid * NODES_PER_TILE, NODES_PER_TILE)])


BP = 512

_L = np.arange(128)
_M_DEG = (_L[:, None] == (_L[None, :] // 16) * 16 + 8).astype(np.float32)
_SEL0 = (_L[:, None] == (_L[None, :] // 16) * 16).astype(np.float32)
_SEL1 = (_L[:, None] == (_L[None, :] // 16) * 16 + 1).astype(np.float32)
_K16 = np.arange(16)
_C_OUT = (_L[:, None] == 16 * (_K16[None, :] // 2) + _K16[None, :] % 2
          ).astype(np.float32)
_SPREAD = np.kron(np.eye(8), np.hstack([np.eye(8), np.zeros((8, 8))])
                  ).astype(np.float32)
_ONES_B = ((_L % 16) == 8).astype(np.float32)[None, :]


def _dot(a, b):
    return jax.lax.dot_general(a, b, (((1,), (0,)), ((), ())),
                               preferred_element_type=jnp.float32)


def _valid_rows(i):
    row = i * BP + lax.broadcasted_iota(jnp.int32, (BP, 1), 0)
    return row < NV128


def _prep_body(x_ref, spread, ones_b, g1_ref):
    i = pl.program_id(0)
    g1 = _dot(x_ref[...], spread[...]) + ones_b[...]
    g1_ref[...] = jnp.where(_valid_rows(i), g1, 0.0)


def _dense1_body(p_ref, x_ref, mdeg, kwl1, kwr1, b1p, kwl2a, kwl2b, kwr2a,
                 kwr2b, g2a_ref, g2b_ref, r2a_ref, r2b_ref, dinv_ref):
    i = pl.program_id(0)
    p = p_ref[...]
    agg = p[0] + p[1]
    deg_b = _dot(agg, mdeg[...])
    dinv = 1.0 / jnp.maximum(deg_b, 1.0)
    mean = agg * dinv
    h1 = jnp.maximum(_dot(mean, kwl1[...]) + _dot(x_ref[...], kwr1[...])
                     + b1p[...], 0.0)
    valid = _valid_rows(i)
    g2a_ref[...] = jnp.where(valid, _dot(h1, kwl2a[...]), 0.0)
    g2b_ref[...] = jnp.where(valid, _dot(h1, kwl2b[...]), 0.0)
    r2a_ref[...] = _dot(h1, kwr2a[...])
    r2b_ref[...] = _dot(h1, kwr2b[...])
    dinv_ref[...] = dinv


def _dense2_body(p2a_ref, p2b_ref, r2a_ref, r2b_ref, dinv_ref, b2ap, b2bp,
                 kwl3a, kwl3b, kwr3a, kwr3b, g3_ref, r3_ref):
    i = pl.program_id(0)
    p2a = p2a_ref[...]
    p2b = p2b_ref[...]
    dinv = dinv_ref[...]
    h2a = jnp.maximum((p2a[0] + p2a[1]) * dinv + b2ap[...] + r2a_ref[...], 0.0)
    h2b = jnp.maximum((p2b[0] + p2b[1]) * dinv + b2bp[...] + r2b_ref[...], 0.0)
    g3 = _dot(h2a, kwl3a[...]) + _dot(h2b, kwl3b[...])
    g3_ref[...] = jnp.where(_valid_rows(i), g3, 0.0)
    r3_ref[...] = _dot(h2a, kwr3a[...]) + _dot(h2b, kwr3b[...])


def _dense3_body(p3_ref, r3_ref, dinv_ref, b3p, kfc1, f1bp, kfc2, f2bp,
                 sel0, sel1, cout, out_ref):
    p3 = p3_ref[...]
    h3 = jnp.maximum((p3[0] + p3[1]) * dinv_ref[...] + b3p[...] + r3_ref[...],
                     0.0)
    f1 = jnp.maximum(_dot(h3, kfc1[...]) + f1bp[...], 0.0)
    lg = _dot(f1, kfc2[...]) + f2bp[...]
    l0 = _dot(lg, sel0[...])
    l1 = _dot(lg, sel1[...])
    m = jnp.maximum(l0, l1)
    lse = m + jnp.log(jnp.exp(l0 - m) + jnp.exp(l1 - m))
    out_ref[...] = _dot(lg - lse, cout[...])


def _blk(width):
    return pl.BlockSpec((BP, width), lambda i: (i, 0))


def _p_spec():
    return pl.BlockSpec((2, BP, 128), lambda i: (0, i, 0))


def _w_spec(shape):
    return pl.BlockSpec(shape, lambda i: tuple(0 for _ in shape))


def _shape(width):
    return jax.ShapeDtypeStruct((NP128, width), jnp.float32)


def kernel(x, edge_index, Wl1, bl1, Wr1, Wl2, bl2, Wr2, Wl3, bl3, Wr3,
           fc1_w, fc1_b, fc2_w, fc2_b):
    n = x.shape[0]
    e = edge_index.shape[1]
    assert n == N_NODES and e == N_EDGES
    f32 = jnp.float32

    edges2d = jnp.pad(edge_index, ((0, 0), (0, E_PAD - e)),
                      constant_values=n).reshape(2, IDX_ROWS, 128)

    x64 = jnp.pad(x, ((0, N_PAD - n), (0, 0))).reshape(NP128, 64)

    i8 = jnp.eye(8, dtype=f32)
    kwr1 = jnp.kron(i8, Wr1.T)
    kwl1 = jnp.kron(i8, jnp.pad(Wl1.T, ((0, 8), (0, 0))))
    kwl2a = jnp.kron(i8, Wl2.T[:, 0:16])
    kwl2b = jnp.kron(i8, Wl2.T[:, 16:32])
    kwr2a = jnp.kron(i8, Wr2.T[:, 0:16])
    kwr2b = jnp.kron(i8, Wr2.T[:, 16:32])
    kwl3a = jnp.kron(i8, Wl3.T[0:16])
    kwl3b = jnp.kron(i8, Wl3.T[16:32])
    kwr3a = jnp.kron(i8, Wr3.T[0:16])
    kwr3b = jnp.kron(i8, Wr3.T[16:32])
    kfc1 = jnp.kron(i8, jnp.pad(fc1_w.T, ((0, 0), (0, 8))))
    kfc2 = jnp.kron(i8, jnp.pad(fc2_w.T, ((0, 8), (0, 14))))
    b1p = jnp.tile(bl1, 8).reshape(1, 512)
    b2ap = jnp.tile(bl2[0:16], 8).reshape(1, 128)
    b2bp = jnp.tile(bl2[16:32], 8).reshape(1, 128)
    b3p = jnp.tile(bl3, 8).reshape(1, 128)
    f1bp = jnp.tile(jnp.pad(fc1_b, (0, 8)), 8).reshape(1, 128)
    f2bp = jnp.tile(jnp.pad(fc2_b, (0, 14)), 8).reshape(1, 128)

    grid = (NP128 // BP,)

    g1 = pl.pallas_call(
        _prep_body, grid=grid,
        in_specs=[_blk(64), _w_spec((64, 128)), _w_spec((1, 128))],
        out_specs=[_blk(128)],
        out_shape=[_shape(128)],
    )(x64, jnp.asarray(_SPREAD), jnp.asarray(_ONES_B))[0]
    p1 = _sc_segment_sum(g1, edges2d)

    g2a, g2b, r2a, r2b, dinv = pl.pallas_call(
        _dense1_body, grid=grid,
        in_specs=[_p_spec(), _blk(64), _w_spec((128, 128)),
                  _w_spec((128, 512)), _w_spec((64, 512)),
                  _w_spec((1, 512))] + [_w_spec((512, 128))] * 4,
        out_specs=[_blk(128)] * 5,
        out_shape=[_shape(128)] * 5,
    )(p1, x64, jnp.asarray(_M_DEG), kwl1, kwr1, b1p, kwl2a, kwl2b, kwr2a,
      kwr2b)

    p2a = _sc_segment_sum(g2a, edges2d)
    p2b = _sc_segment_sum(g2b, edges2d)

    g3, r3 = pl.pallas_call(
        _dense2_body, grid=grid,
        in_specs=[_p_spec(), _p_spec(), _blk(128), _blk(128), _blk(128),
                  _w_spec((1, 128)), _w_spec((1, 128)),
                  _w_spec((128, 128)), _w_spec((128, 128)),
                  _w_spec((128, 128)), _w_spec((128, 128))],
        out_specs=[_blk(128)] * 2,
        out_shape=[_shape(128)] * 2,
    )(p2a, p2b, r2a, r2b, dinv, b2ap, b2bp, kwl3a, kwl3b, kwr3a, kwr3b)

    p3 = _sc_segment_sum(g3, edges2d)

    out16 = pl.pallas_call(
        _dense3_body, grid=grid,
        in_specs=[_p_spec(), _blk(128), _blk(128), _w_spec((1, 128)),
                  _w_spec((128, 128)), _w_spec((1, 128)),
                  _w_spec((128, 128)), _w_spec((1, 128)),
                  _w_spec((128, 128)), _w_spec((128, 128)),
                  _w_spec((128, 16))],
        out_specs=[_blk(16)],
        out_shape=[_shape(16)],
    )(p3, r3, dinv, b3p, kfc1, f1bp, kfc2, f2bp,
      jnp.asarray(_SEL0), jnp.asarray(_SEL1), jnp.asarray(_C_OUT))[0]

    return out16.reshape(N_PAD, 2)[:N_NODES]


# speedup vs baseline: 24.6559x; 1.2296x over previous
import functools

import jax
import jax.numpy as jnp
import numpy as np
from jax import lax
from jax.experimental import pallas as pl
from jax.experimental.pallas import tpu as pltpu
from jax.experimental.pallas import tpu_sc as plsc

N_NODES = 100000
N_EDGES = 1600000

NC = 2
NS = 16
NW = NC * NS
CHUNK = 512
STREAMS = 4
CHUNKS_PER_TILE = 98
EDGES_PER_TILE = CHUNKS_PER_TILE * CHUNK
E_PAD = NW * EDGES_PER_TILE
IDX_ROWS = E_PAD // 128
ROWS_PER_TILE_IDX = EDGES_PER_TILE // 128
N_PAD = 102400
NODES_PER_TILE = N_PAD // NS
D = 16
NP128 = N_PAD * D // 128
NV128 = N_NODES * D // 128


@functools.cache
def _build_sc_segment_sum():
    mesh = plsc.VectorSubcoreMesh(
        core_axis_name="c", subcore_axis_name="s",
        num_cores=NC, num_subcores=NS)
    return functools.partial(
        pl.kernel,
        out_type=jax.ShapeDtypeStruct((NC, N_PAD, D), jnp.float32),
        mesh=mesh,
        scratch_types=[
            pltpu.VMEM((STREAMS, 128), jnp.int32),
            pltpu.VMEM((STREAMS, 128), jnp.int32),
            pltpu.VMEM((STREAMS, 128), jnp.int32),
            pltpu.VMEM((STREAMS, 128), jnp.int32),
            pltpu.VMEM((CHUNK, D), jnp.float32),
            pltpu.VMEM((CHUNK, D), jnp.float32),
            pltpu.VMEM_SHARED((N_PAD, D), jnp.float32),
            pltpu.SemaphoreType.DMA,
            pltpu.SemaphoreType.DMA,
            pltpu.SemaphoreType.DMA,
            pltpu.SemaphoreType.DMA,
        ],
        compiler_params=pltpu.CompilerParams(use_tc_tiling_on_sc=False),
    )(_sc_segment_sum_body)


def _sc_segment_sum(g128, edges2d):
    p = _build_sc_segment_sum()(g128.reshape(N_PAD, D), edges2d)
    return p.reshape(NC, NP128, 128)


def _sc_segment_sum_body(g_hbm, edges_hbm, out_hbm,
                         idx_a, idx_b, dst_a, dst_b, rows_a, rows_b,
                         acc, sem_ga, sem_gb, sem_sa, sem_sb):
    cid = lax.axis_index("c")
    sid = lax.axis_index("s")
    wid = cid * NS + sid

    def zero_body(i, carry):
        rows_a[i] = jnp.zeros((D,), jnp.float32)
        return carry
    lax.fori_loop(0, CHUNK, zero_body, 0)
    base = sid * NODES_PER_TILE
    for t in range(NODES_PER_TILE // CHUNK):
        pltpu.sync_copy(rows_a, acc.at[pl.ds(base + t * CHUNK, CHUNK)])
    rem = NODES_PER_TILE % CHUNK
    if rem:
        pltpu.sync_copy(rows_a.at[pl.ds(0, rem)],
                        acc.at[pl.ds(base + NODES_PER_TILE - rem, rem)])
    plsc.subcore_barrier()

    row_base = wid * ROWS_PER_TILE_IDX

    def fire_gathers(idx_v, rows_v, sem, r0):
        pltpu.sync_copy(edges_hbm.at[0, pl.ds(r0, STREAMS)], idx_v)
        for j in range(STREAMS):
            pltpu.async_copy(g_hbm.at[idx_v.at[j]],
                             rows_v.at[pl.ds(j * 128, 128)], sem)

    def fire_scatters(dst_v, rows_v, sem, r0):
        pltpu.sync_copy(edges_hbm.at[1, pl.ds(r0, STREAMS)], dst_v)
        for j in range(STREAMS):
            pltpu.async_copy(rows_v.at[pl.ds(j * 128, 128)],
                             acc.at[dst_v.at[j]], sem, add=True)

    def drain_gathers(idx_v, rows_v, sem):
        for j in range(STREAMS):
            pltpu.make_async_copy(g_hbm.at[idx_v.at[j]],
                                  rows_v.at[pl.ds(j * 128, 128)], sem).wait()

    def drain_scatters(dst_v, rows_v, sem):
        for j in range(STREAMS):
            pltpu.make_async_copy(rows_v.at[pl.ds(j * 128, 128)],
                                  acc.at[dst_v.at[j]], sem).wait()

    fire_gathers(idx_a, rows_a, sem_ga, row_base)

    def pair_body(k, carry):
        r0 = row_base + 2 * k * STREAMS

        @pl.when(k > 0)
        def _():
            drain_scatters(dst_b, rows_b, sem_sb)
        fire_gathers(idx_b, rows_b, sem_gb, r0 + STREAMS)
        drain_gathers(idx_a, rows_a, sem_ga)
        fire_scatters(dst_a, rows_a, sem_sa, r0)
        drain_gathers(idx_b, rows_b, sem_gb)
        fire_scatters(dst_b, rows_b, sem_sb, r0 + STREAMS)
        drain_scatters(dst_a, rows_a, sem_sa)

        @pl.when(k < CHUNKS_PER_TILE // 2 - 1)
        def _():
            fire_gathers(idx_a, rows_a, sem_ga, r0 + 2 * STREAMS)
        return carry
    lax.fori_loop(0, CHUNKS_PER_TILE // 2, pair_body, 0)
    drain_scatters(dst_b, rows_b, sem_sb)
    plsc.subcore_barrier()

    pltpu.sync_copy(acc.at[pl.ds(sid * NODES_PER_TILE, NODES_PER_TILE)],
                    out_hbm.at[cid, pl.ds(sid * NODES_PER_TILE, NODES_PER_TILE)])


BP = 512

_L = np.arange(128)
_M_DEG = (_L[:, None] == (_L[None, :] // 16) * 16 + 8).astype(np.float32)
_SEL0 = (_L[:, None] == (_L[None, :] // 16) * 16).astype(np.float32)
_SEL1 = (_L[:, None] == (_L[None, :] // 16) * 16 + 1).astype(np.float32)
_K16 = np.arange(16)
_C_OUT = (_L[:, None] == 16 * (_K16[None, :] // 2) + _K16[None, :] % 2
          ).astype(np.float32)
_SPREAD = np.kron(np.eye(8), np.hstack([np.eye(8), np.zeros((8, 8))])
                  ).astype(np.float32)
_ONES_B = ((_L % 16) == 8).astype(np.float32)[None, :]


def _dot(a, b):
    return jax.lax.dot_general(a, b, (((1,), (0,)), ((), ())),
                               preferred_element_type=jnp.float32)


def _valid_rows(i):
    row = i * BP + lax.broadcasted_iota(jnp.int32, (BP, 1), 0)
    return row < NV128


def _prep_body(x_ref, spread, ones_b, g1_ref):
    i = pl.program_id(0)
    g1 = _dot(x_ref[...], spread[...]) + ones_b[...]
    g1_ref[...] = jnp.where(_valid_rows(i), g1, 0.0)


def _dense1_body(p_ref, x_ref, mdeg, kwl1, kwr1, b1p, kwl2a, kwl2b, kwr2a,
                 kwr2b, g2a_ref, g2b_ref, r2a_ref, r2b_ref, dinv_ref):
    i = pl.program_id(0)
    p = p_ref[...]
    agg = p[0] + p[1]
    deg_b = _dot(agg, mdeg[...])
    dinv = 1.0 / jnp.maximum(deg_b, 1.0)
    mean = agg * dinv
    h1 = jnp.maximum(_dot(mean, kwl1[...]) + _dot(x_ref[...], kwr1[...])
                     + b1p[...], 0.0)
    valid = _valid_rows(i)
    g2a_ref[...] = jnp.where(valid, _dot(h1, kwl2a[...]), 0.0)
    g2b_ref[...] = jnp.where(valid, _dot(h1, kwl2b[...]), 0.0)
    r2a_ref[...] = _dot(h1, kwr2a[...])
    r2b_ref[...] = _dot(h1, kwr2b[...])
    dinv_ref[...] = dinv


def _dense2_body(p2a_ref, p2b_ref, r2a_ref, r2b_ref, dinv_ref, b2ap, b2bp,
                 kwl3a, kwl3b, kwr3a, kwr3b, g3_ref, r3_ref):
    i = pl.program_id(0)
    p2a = p2a_ref[...]
    p2b = p2b_ref[...]
    dinv = dinv_ref[...]
    h2a = jnp.maximum((p2a[0] + p2a[1]) * dinv + b2ap[...] + r2a_ref[...], 0.0)
    h2b = jnp.maximum((p2b[0] + p2b[1]) * dinv + b2bp[...] + r2b_ref[...], 0.0)
    g3 = _dot(h2a, kwl3a[...]) + _dot(h2b, kwl3b[...])
    g3_ref[...] = jnp.where(_valid_rows(i), g3, 0.0)
    r3_ref[...] = _dot(h2a, kwr3a[...]) + _dot(h2b, kwr3b[...])


def _dense3_body(p3_ref, r3_ref, dinv_ref, b3p, kfc1, f1bp, kfc2, f2bp,
                 sel0, sel1, cout, out_ref):
    p3 = p3_ref[...]
    h3 = jnp.maximum((p3[0] + p3[1]) * dinv_ref[...] + b3p[...] + r3_ref[...],
                     0.0)
    f1 = jnp.maximum(_dot(h3, kfc1[...]) + f1bp[...], 0.0)
    lg = _dot(f1, kfc2[...]) + f2bp[...]
    l0 = _dot(lg, sel0[...])
    l1 = _dot(lg, sel1[...])
    m = jnp.maximum(l0, l1)
    lse = m + jnp.log(jnp.exp(l0 - m) + jnp.exp(l1 - m))
    out_ref[...] = _dot(lg - lse, cout[...])


def _blk(width):
    return pl.BlockSpec((BP, width), lambda i: (i, 0))


def _p_spec():
    return pl.BlockSpec((2, BP, 128), lambda i: (0, i, 0))


def _w_spec(shape):
    return pl.BlockSpec(shape, lambda i: tuple(0 for _ in shape))


def _shape(width):
    return jax.ShapeDtypeStruct((NP128, width), jnp.float32)


def kernel(x, edge_index, Wl1, bl1, Wr1, Wl2, bl2, Wr2, Wl3, bl3, Wr3,
           fc1_w, fc1_b, fc2_w, fc2_b):
    n = x.shape[0]
    e = edge_index.shape[1]
    assert n == N_NODES and e == N_EDGES
    f32 = jnp.float32

    edges2d = jnp.pad(edge_index, ((0, 0), (0, E_PAD - e)),
                      constant_values=n).reshape(2, IDX_ROWS, 128)

    x64 = jnp.pad(x, ((0, N_PAD - n), (0, 0))).reshape(NP128, 64)

    i8 = jnp.eye(8, dtype=f32)
    kwr1 = jnp.kron(i8, Wr1.T)
    kwl1 = jnp.kron(i8, jnp.pad(Wl1.T, ((0, 8), (0, 0))))
    kwl2a = jnp.kron(i8, Wl2.T[:, 0:16])
    kwl2b = jnp.kron(i8, Wl2.T[:, 16:32])
    kwr2a = jnp.kron(i8, Wr2.T[:, 0:16])
    kwr2b = jnp.kron(i8, Wr2.T[:, 16:32])
    kwl3a = jnp.kron(i8, Wl3.T[0:16])
    kwl3b = jnp.kron(i8, Wl3.T[16:32])
    kwr3a = jnp.kron(i8, Wr3.T[0:16])
    kwr3b = jnp.kron(i8, Wr3.T[16:32])
    kfc1 = jnp.kron(i8, jnp.pad(fc1_w.T, ((0, 0), (0, 8))))
    kfc2 = jnp.kron(i8, jnp.pad(fc2_w.T, ((0, 8), (0, 14))))
    b1p = jnp.tile(bl1, 8).reshape(1, 512)
    b2ap = jnp.tile(bl2[0:16], 8).reshape(1, 128)
    b2bp = jnp.tile(bl2[16:32], 8).reshape(1, 128)
    b3p = jnp.tile(bl3, 8).reshape(1, 128)
    f1bp = jnp.tile(jnp.pad(fc1_b, (0, 8)), 8).reshape(1, 128)
    f2bp = jnp.tile(jnp.pad(fc2_b, (0, 14)), 8).reshape(1, 128)

    grid = (NP128 // BP,)

    g1 = pl.pallas_call(
        _prep_body, grid=grid,
        in_specs=[_blk(64), _w_spec((64, 128)), _w_spec((1, 128))],
        out_specs=[_blk(128)],
        out_shape=[_shape(128)],
    )(x64, jnp.asarray(_SPREAD), jnp.asarray(_ONES_B))[0]
    p1 = _sc_segment_sum(g1, edges2d)

    g2a, g2b, r2a, r2b, dinv = pl.pallas_call(
        _dense1_body, grid=grid,
        in_specs=[_p_spec(), _blk(64), _w_spec((128, 128)),
                  _w_spec((128, 512)), _w_spec((64, 512)),
                  _w_spec((1, 512))] + [_w_spec((512, 128))] * 4,
        out_specs=[_blk(128)] * 5,
        out_shape=[_shape(128)] * 5,
    )(p1, x64, jnp.asarray(_M_DEG), kwl1, kwr1, b1p, kwl2a, kwl2b, kwr2a,
      kwr2b)

    p2a = _sc_segment_sum(g2a, edges2d)
    p2b = _sc_segment_sum(g2b, edges2d)

    g3, r3 = pl.pallas_call(
        _dense2_body, grid=grid,
        in_specs=[_p_spec(), _p_spec(), _blk(128), _blk(128), _blk(128),
                  _w_spec((1, 128)), _w_spec((1, 128)),
                  _w_spec((128, 128)), _w_spec((128, 128)),
                  _w_spec((128, 128)), _w_spec((128, 128))],
        out_specs=[_blk(128)] * 2,
        out_shape=[_shape(128)] * 2,
    )(p2a, p2b, r2a, r2b, dinv, b2ap, b2bp, kwl3a, kwl3b, kwr3a, kwr3b)

    p3 = _sc_segment_sum(g3, edges2d)

    out16 = pl.pallas_call(
        _dense3_body, grid=grid,
        in_specs=[_p_spec(), _blk(128), _blk(128), _w_spec((1, 128)),
                  _w_spec((128, 128)), _w_spec((1, 128)),
                  _w_spec((128, 128)), _w_spec((1, 128)),
                  _w_spec((128, 128)), _w_spec((128, 128)),
                  _w_spec((128, 16))],
        out_specs=[_blk(16)],
        out_shape=[_shape(16)],
    )(p3, r3, dinv, b3p, kfc1, f1bp, kfc2, f2bp,
      jnp.asarray(_SEL0), jnp.asarray(_SEL1), jnp.asarray(_C_OUT))[0]

    return out16.reshape(N_PAD, 2)[:N_NODES]


# speedup vs baseline: 25.3378x; 1.0277x over previous
import functools

import jax
import jax.numpy as jnp
import numpy as np
from jax import lax
from jax.experimental import pallas as pl
from jax.experimental.pallas import tpu as pltpu
from jax.experimental.pallas import tpu_sc as plsc

N_NODES = 100000
N_EDGES = 1600000

NC = 2
NS = 16
NW = NC * NS
CHUNK = 512
STREAMS = 4
CHUNKS_PER_TILE = 98
EDGES_PER_TILE = CHUNKS_PER_TILE * CHUNK
E_PAD = NW * EDGES_PER_TILE
IDX_ROWS = E_PAD // 128
ROWS_PER_TILE_IDX = EDGES_PER_TILE // 128
N_PAD = 102400
NODES_PER_TILE = N_PAD // NS
D = 16
NP128 = N_PAD * D // 128
NV128 = N_NODES * D // 128


@functools.cache
def _build_sc_segment_sum():
    mesh = plsc.VectorSubcoreMesh(
        core_axis_name="c", subcore_axis_name="s",
        num_cores=NC, num_subcores=NS)
    return functools.partial(
        pl.kernel,
        out_type=jax.ShapeDtypeStruct((NC, N_PAD, D), jnp.float32),
        mesh=mesh,
        scratch_types=[
            pltpu.VMEM((STREAMS, 128), jnp.int32),
            pltpu.VMEM((STREAMS, 128), jnp.int32),
            pltpu.VMEM((STREAMS, 128), jnp.int32),
            pltpu.VMEM((STREAMS, 128), jnp.int32),
            pltpu.VMEM((CHUNK, D), jnp.float32),
            pltpu.VMEM((CHUNK, D), jnp.float32),
            pltpu.VMEM_SHARED((N_PAD, D), jnp.float32),
            pltpu.SemaphoreType.DMA,
            pltpu.SemaphoreType.DMA,
            pltpu.SemaphoreType.DMA,
            pltpu.SemaphoreType.DMA,
        ],
        compiler_params=pltpu.CompilerParams(use_tc_tiling_on_sc=False),
    )(_sc_segment_sum_body)


def _sc_segment_sum(g128, edges2d):
    p = _build_sc_segment_sum()(g128.reshape(N_PAD, D), edges2d)
    return p.reshape(NC, NP128, 128)


def _sc_segment_sum_body(g_hbm, edges_hbm, out_hbm,
                         idx_a, idx_b, dst_a, dst_b, rows_a, rows_b,
                         acc, sem_ga, sem_gb, sem_sa, sem_sb):
    cid = lax.axis_index("c")
    sid = lax.axis_index("s")
    wid = cid * NS + sid

    def zero_body(i, carry):
        rows_a[i] = jnp.zeros((D,), jnp.float32)
        return carry
    lax.fori_loop(0, CHUNK, zero_body, 0)
    base = sid * NODES_PER_TILE
    for t in range(NODES_PER_TILE // CHUNK):
        pltpu.sync_copy(rows_a, acc.at[pl.ds(base + t * CHUNK, CHUNK)])
    rem = NODES_PER_TILE % CHUNK
    if rem:
        pltpu.sync_copy(rows_a.at[pl.ds(0, rem)],
                        acc.at[pl.ds(base + NODES_PER_TILE - rem, rem)])
    plsc.subcore_barrier()

    row_base = wid * ROWS_PER_TILE_IDX

    def fire_gathers(idx_v, rows_v, sem, r0):
        pltpu.sync_copy(edges_hbm.at[0, pl.ds(r0, STREAMS)], idx_v)
        for j in range(STREAMS):
            pltpu.async_copy(g_hbm.at[idx_v.at[j]],
                             rows_v.at[pl.ds(j * 128, 128)], sem)

    def fire_scatters(dst_v, rows_v, sem, r0):
        pltpu.sync_copy(edges_hbm.at[1, pl.ds(r0, STREAMS)], dst_v)
        for j in range(STREAMS):
            pltpu.async_copy(rows_v.at[pl.ds(j * 128, 128)],
                             acc.at[dst_v.at[j]], sem, add=True)

    def drain_gathers(idx_v, rows_v, sem):
        for j in range(STREAMS):
            pltpu.make_async_copy(g_hbm.at[idx_v.at[j]],
                                  rows_v.at[pl.ds(j * 128, 128)], sem).wait()

    def drain_scatters(dst_v, rows_v, sem):
        for j in range(STREAMS):
            pltpu.make_async_copy(rows_v.at[pl.ds(j * 128, 128)],
                                  acc.at[dst_v.at[j]], sem).wait()

    fire_gathers(idx_a, rows_a, sem_ga, row_base)

    def pair_body(k, carry):
        r0 = row_base + 2 * k * STREAMS

        @pl.when(k > 0)
        def _():
            drain_scatters(dst_b, rows_b, sem_sb)
        fire_gathers(idx_b, rows_b, sem_gb, r0 + STREAMS)
        drain_gathers(idx_a, rows_a, sem_ga)
        fire_scatters(dst_a, rows_a, sem_sa, r0)
        drain_gathers(idx_b, rows_b, sem_gb)
        fire_scatters(dst_b, rows_b, sem_sb, r0 + STREAMS)
        drain_scatters(dst_a, rows_a, sem_sa)

        @pl.when(k < CHUNKS_PER_TILE // 2 - 1)
        def _():
            fire_gathers(idx_a, rows_a, sem_ga, r0 + 2 * STREAMS)
        return carry
    lax.fori_loop(0, CHUNKS_PER_TILE // 2, pair_body, 0)
    drain_scatters(dst_b, rows_b, sem_sb)
    plsc.subcore_barrier()

    pltpu.sync_copy(acc.at[pl.ds(sid * NODES_PER_TILE, NODES_PER_TILE)],
                    out_hbm.at[cid, pl.ds(sid * NODES_PER_TILE, NODES_PER_TILE)])


BP = 512

_L = np.arange(128)
_M_DEG = (_L[:, None] == (_L[None, :] // 16) * 16 + 8).astype(np.float32)
_SEL0 = (_L[:, None] == (_L[None, :] // 16) * 16).astype(np.float32)
_SEL1 = (_L[:, None] == (_L[None, :] // 16) * 16 + 1).astype(np.float32)
_K16 = np.arange(16)
_C_OUT = (_L[:, None] == 16 * (_K16[None, :] // 2) + _K16[None, :] % 2
          ).astype(np.float32)
_SPREAD = np.kron(np.eye(8), np.hstack([np.eye(8), np.zeros((8, 8))])
                  ).astype(np.float32)
_ONES_B = ((_L % 16) == 8).astype(np.float32)[None, :]


def _dot(a, b):
    return jax.lax.dot_general(a, b, (((1,), (0,)), ((), ())),
                               preferred_element_type=jnp.float32)


def _valid_rows(i):
    row = i * BP + lax.broadcasted_iota(jnp.int32, (BP, 1), 0)
    return row < NV128


def _prep_body(x_ref, spread, ones_b, g1_ref):
    i = pl.program_id(0)
    g1 = _dot(x_ref[...], spread[...]) + ones_b[...]
    g1_ref[...] = jnp.where(_valid_rows(i), g1, 0.0)


def _dense1_body(p_ref, x_ref, mdeg, kwl1, kwr1, b1p, kwl2a, kwl2b, kwr2a,
                 kwr2b, g2a_ref, g2b_ref, r2a_ref, r2b_ref, dinv_ref):
    i = pl.program_id(0)
    p = p_ref[...]
    agg = p[0] + p[1]
    deg_b = _dot(agg, mdeg[...])
    dinv = 1.0 / jnp.maximum(deg_b, 1.0)
    mean = agg * dinv
    h1 = jnp.maximum(_dot(mean, kwl1[...]) + _dot(x_ref[...], kwr1[...])
                     + b1p[...], 0.0)
    valid = _valid_rows(i)
    g2a_ref[...] = jnp.where(valid, _dot(h1, kwl2a[...]), 0.0)
    g2b_ref[...] = jnp.where(valid, _dot(h1, kwl2b[...]), 0.0)
    r2a_ref[...] = _dot(h1, kwr2a[...])
    r2b_ref[...] = _dot(h1, kwr2b[...])
    dinv_ref[...] = dinv


def _dense2_body(p2a_ref, p2b_ref, r2a_ref, r2b_ref, dinv_ref, b2ap, b2bp,
                 kwl3a, kwl3b, kwr3a, kwr3b, g3_ref, r3_ref):
    i = pl.program_id(0)
    p2a = p2a_ref[...]
    p2b = p2b_ref[...]
    dinv = dinv_ref[...]
    h2a = jnp.maximum((p2a[0] + p2a[1]) * dinv + b2ap[...] + r2a_ref[...], 0.0)
    h2b = jnp.maximum((p2b[0] + p2b[1]) * dinv + b2bp[...] + r2b_ref[...], 0.0)
    g3 = _dot(h2a, kwl3a[...]) + _dot(h2b, kwl3b[...])
    g3_ref[...] = jnp.where(_valid_rows(i), g3, 0.0)
    r3_ref[...] = _dot(h2a, kwr3a[...]) + _dot(h2b, kwr3b[...])


def _dense3_body(p3_ref, r3_ref, dinv_ref, b3p, kfc1, f1bp, kfc2, f2bp,
                 sel0, sel1, cout, out_ref):
    p3 = p3_ref[...]
    h3 = jnp.maximum((p3[0] + p3[1]) * dinv_ref[...] + b3p[...] + r3_ref[...],
                     0.0)
    f1 = jnp.maximum(_dot(h3, kfc1[...]) + f1bp[...], 0.0)
    lg = _dot(f1, kfc2[...]) + f2bp[...]
    l0 = _dot(lg, sel0[...])
    l1 = _dot(lg, sel1[...])
    m = jnp.maximum(l0, l1)
    lse = m + jnp.log(jnp.exp(l0 - m) + jnp.exp(l1 - m))
    out_ref[...] = _dot(lg - lse, cout[...])


def _blk(width):
    return pl.BlockSpec((BP, width), lambda i: (i, 0))


def _p_spec():
    return pl.BlockSpec((2, BP, 128), lambda i: (0, i, 0))


def _w_spec(shape):
    return pl.BlockSpec(shape, lambda i: tuple(0 for _ in shape))


def _shape(width):
    return jax.ShapeDtypeStruct((NP128, width), jnp.float32)


def kernel(x, edge_index, Wl1, bl1, Wr1, Wl2, bl2, Wr2, Wl3, bl3, Wr3,
           fc1_w, fc1_b, fc2_w, fc2_b):
    n = x.shape[0]
    e = edge_index.shape[1]
    assert n == N_NODES and e == N_EDGES
    f32 = jnp.float32

    edges2d = jnp.pad(edge_index, ((0, 0), (0, E_PAD - e)),
                      constant_values=n).reshape(2, IDX_ROWS, 128)

    x64 = jnp.pad(x.reshape(NV128, 64), ((0, NP128 - NV128), (0, 0)))

    i8 = jnp.eye(8, dtype=f32)
    kwr1 = jnp.kron(i8, Wr1.T)
    kwl1 = jnp.kron(i8, jnp.pad(Wl1.T, ((0, 8), (0, 0))))
    kwl2a = jnp.kron(i8, Wl2.T[:, 0:16])
    kwl2b = jnp.kron(i8, Wl2.T[:, 16:32])
    kwr2a = jnp.kron(i8, Wr2.T[:, 0:16])
    kwr2b = jnp.kron(i8, Wr2.T[:, 16:32])
    kwl3a = jnp.kron(i8, Wl3.T[0:16])
    kwl3b = jnp.kron(i8, Wl3.T[16:32])
    kwr3a = jnp.kron(i8, Wr3.T[0:16])
    kwr3b = jnp.kron(i8, Wr3.T[16:32])
    kfc1 = jnp.kron(i8, jnp.pad(fc1_w.T, ((0, 0), (0, 8))))
    kfc2 = jnp.kron(i8, jnp.pad(fc2_w.T, ((0, 8), (0, 14))))
    b1p = jnp.tile(bl1, 8).reshape(1, 512)
    b2ap = jnp.tile(bl2[0:16], 8).reshape(1, 128)
    b2bp = jnp.tile(bl2[16:32], 8).reshape(1, 128)
    b3p = jnp.tile(bl3, 8).reshape(1, 128)
    f1bp = jnp.tile(jnp.pad(fc1_b, (0, 8)), 8).reshape(1, 128)
    f2bp = jnp.tile(jnp.pad(fc2_b, (0, 14)), 8).reshape(1, 128)

    grid = (NP128 // BP,)

    g1 = pl.pallas_call(
        _prep_body, grid=grid,
        in_specs=[_blk(64), _w_spec((64, 128)), _w_spec((1, 128))],
        out_specs=[_blk(128)],
        out_shape=[_shape(128)],
    )(x64, jnp.asarray(_SPREAD), jnp.asarray(_ONES_B))[0]
    p1 = _sc_segment_sum(g1, edges2d)

    g2a, g2b, r2a, r2b, dinv = pl.pallas_call(
        _dense1_body, grid=grid,
        in_specs=[_p_spec(), _blk(64), _w_spec((128, 128)),
                  _w_spec((128, 512)), _w_spec((64, 512)),
                  _w_spec((1, 512))] + [_w_spec((512, 128))] * 4,
        out_specs=[_blk(128)] * 5,
        out_shape=[_shape(128)] * 5,
    )(p1, x64, jnp.asarray(_M_DEG), kwl1, kwr1, b1p, kwl2a, kwl2b, kwr2a,
      kwr2b)

    p2a = _sc_segment_sum(g2a, edges2d)
    p2b = _sc_segment_sum(g2b, edges2d)

    g3, r3 = pl.pallas_call(
        _dense2_body, grid=grid,
        in_specs=[_p_spec(), _p_spec(), _blk(128), _blk(128), _blk(128),
                  _w_spec((1, 128)), _w_spec((1, 128)),
                  _w_spec((128, 128)), _w_spec((128, 128)),
                  _w_spec((128, 128)), _w_spec((128, 128))],
        out_specs=[_blk(128)] * 2,
        out_shape=[_shape(128)] * 2,
    )(p2a, p2b, r2a, r2b, dinv, b2ap, b2bp, kwl3a, kwl3b, kwr3a, kwr3b)

    p3 = _sc_segment_sum(g3, edges2d)

    out16 = pl.pallas_call(
        _dense3_body, grid=grid,
        in_specs=[_p_spec(), _blk(128), _blk(128), _w_spec((1, 128)),
                  _w_spec((128, 128)), _w_spec((1, 128)),
                  _w_spec((128, 128)), _w_spec((1, 128)),
                  _w_spec((128, 128)), _w_spec((128, 128)),
                  _w_spec((128, 16))],
        out_specs=[_blk(16)],
        out_shape=[_shape(16)],
    )(p3, r3, dinv, b3p, kfc1, f1bp, kfc2, f2bp,
      jnp.asarray(_SEL0), jnp.asarray(_SEL1), jnp.asarray(_C_OUT))[0]

    return out16[:NV128].reshape(N_NODES, 2)


# speedup vs baseline: 25.9849x; 1.0255x over previous
import functools

import jax
import jax.numpy as jnp
import numpy as np
from jax import lax
from jax.experimental import pallas as pl
from jax.experimental.pallas import tpu as pltpu
from jax.experimental.pallas import tpu_sc as plsc

N_NODES = 100000
N_EDGES = 1600000

NC = 2
NS = 16
NW = NC * NS
CHUNK = 512
STREAMS = 4
CHUNKS_PER_TILE = 98
EDGES_PER_TILE = CHUNKS_PER_TILE * CHUNK
E_PAD = NW * EDGES_PER_TILE
IDX_ROWS = E_PAD // 128
ROWS_PER_TILE_IDX = EDGES_PER_TILE // 128
N_PAD = 102400
NODES_PER_TILE = N_PAD // NS
D = 16
NP128 = N_PAD * D // 128
NV128 = N_NODES * D // 128


@functools.cache
def _build_sc_segment_sum():
    mesh = plsc.VectorSubcoreMesh(
        core_axis_name="c", subcore_axis_name="s",
        num_cores=NC, num_subcores=NS)
    return functools.partial(
        pl.kernel,
        out_type=jax.ShapeDtypeStruct((NC, N_PAD, D), jnp.float32),
        mesh=mesh,
        scratch_types=[
            pltpu.VMEM((STREAMS, 128), jnp.int32),
            pltpu.VMEM((STREAMS, 128), jnp.int32),
            pltpu.VMEM((STREAMS, 128), jnp.int32),
            pltpu.VMEM((STREAMS, 128), jnp.int32),
            pltpu.VMEM((CHUNK, D), jnp.float32),
            pltpu.VMEM((CHUNK, D), jnp.float32),
            pltpu.VMEM_SHARED((N_PAD, D), jnp.float32),
            pltpu.SemaphoreType.DMA,
            pltpu.SemaphoreType.DMA,
            pltpu.SemaphoreType.DMA,
            pltpu.SemaphoreType.DMA,
        ],
        compiler_params=pltpu.CompilerParams(use_tc_tiling_on_sc=False),
    )(_sc_segment_sum_body)


def _sc_segment_sum(g128, edges2d):
    p = _build_sc_segment_sum()(g128.reshape(N_PAD, D), edges2d)
    return p.reshape(NC, NP128, 128)


def _sc_segment_sum_body(g_hbm, edges_hbm, out_hbm,
                         idx_a, idx_b, dst_a, dst_b, rows_a, rows_b,
                         acc, sem_ga, sem_gb, sem_sa, sem_sb):
    cid = lax.axis_index("c")
    sid = lax.axis_index("s")
    wid = cid * NS + sid

    def zero_body(i, carry):
        rows_a[i] = jnp.zeros((D,), jnp.float32)
        return carry
    lax.fori_loop(0, CHUNK, zero_body, 0)
    base = sid * NODES_PER_TILE
    for t in range(NODES_PER_TILE // CHUNK):
        pltpu.sync_copy(rows_a, acc.at[pl.ds(base + t * CHUNK, CHUNK)])
    rem = NODES_PER_TILE % CHUNK
    if rem:
        pltpu.sync_copy(rows_a.at[pl.ds(0, rem)],
                        acc.at[pl.ds(base + NODES_PER_TILE - rem, rem)])
    plsc.subcore_barrier()

    c0_chunks = 104
    c1_chunks = 2 * CHUNKS_PER_TILE - c0_chunks
    c0_rows = NS * c0_chunks * STREAMS
    row_base = jnp.where(cid == 0, sid * (c0_chunks * STREAMS),
                         c0_rows + sid * (c1_chunks * STREAMS))
    pairs = jnp.where(cid == 0, c0_chunks // 2, c1_chunks // 2)

    def fire_gathers(idx_v, rows_v, sem, r0):
        pltpu.sync_copy(edges_hbm.at[0, pl.ds(r0, STREAMS)], idx_v)
        for j in range(STREAMS):
            pltpu.async_copy(g_hbm.at[idx_v.at[j]],
                             rows_v.at[pl.ds(j * 128, 128)], sem)

    def fire_scatters(dst_v, rows_v, sem, r0):
        pltpu.sync_copy(edges_hbm.at[1, pl.ds(r0, STREAMS)], dst_v)
        for j in range(STREAMS):
            pltpu.async_copy(rows_v.at[pl.ds(j * 128, 128)],
                             acc.at[dst_v.at[j]], sem, add=True)

    def drain_gathers(idx_v, rows_v, sem):
        for j in range(STREAMS):
            pltpu.make_async_copy(g_hbm.at[idx_v.at[j]],
                                  rows_v.at[pl.ds(j * 128, 128)], sem).wait()

    def drain_scatters(dst_v, rows_v, sem):
        for j in range(STREAMS):
            pltpu.make_async_copy(rows_v.at[pl.ds(j * 128, 128)],
                                  acc.at[dst_v.at[j]], sem).wait()

    fire_gathers(idx_a, rows_a, sem_ga, row_base)

    def pair_body(k, carry):
        r0 = row_base + 2 * k * STREAMS

        @pl.when(k > 0)
        def _():
            drain_scatters(dst_b, rows_b, sem_sb)
        fire_gathers(idx_b, rows_b, sem_gb, r0 + STREAMS)
        drain_gathers(idx_a, rows_a, sem_ga)
        fire_scatters(dst_a, rows_a, sem_sa, r0)
        drain_gathers(idx_b, rows_b, sem_gb)
        fire_scatters(dst_b, rows_b, sem_sb, r0 + STREAMS)
        drain_scatters(dst_a, rows_a, sem_sa)

        @pl.when(k < pairs - 1)
        def _():
            fire_gathers(idx_a, rows_a, sem_ga, r0 + 2 * STREAMS)
        return carry
    lax.fori_loop(0, pairs, pair_body, 0)
    drain_scatters(dst_b, rows_b, sem_sb)
    plsc.subcore_barrier()

    pltpu.sync_copy(acc.at[pl.ds(sid * NODES_PER_TILE, NODES_PER_TILE)],
                    out_hbm.at[cid, pl.ds(sid * NODES_PER_TILE, NODES_PER_TILE)])


BP = 512

_L = np.arange(128)
_M_DEG = (_L[:, None] == (_L[None, :] // 16) * 16 + 8).astype(np.float32)
_SEL0 = (_L[:, None] == (_L[None, :] // 16) * 16).astype(np.float32)
_SEL1 = (_L[:, None] == (_L[None, :] // 16) * 16 + 1).astype(np.float32)
_K16 = np.arange(16)
_C_OUT = (_L[:, None] == 16 * (_K16[None, :] // 2) + _K16[None, :] % 2
          ).astype(np.float32)
_SPREAD = np.kron(np.eye(8), np.hstack([np.eye(8), np.zeros((8, 8))])
                  ).astype(np.float32)
_ONES_B = ((_L % 16) == 8).astype(np.float32)[None, :]


def _dot(a, b):
    return jax.lax.dot_general(a, b, (((1,), (0,)), ((), ())),
                               preferred_element_type=jnp.float32)


def _valid_rows(i):
    row = i * BP + lax.broadcasted_iota(jnp.int32, (BP, 1), 0)
    return row < NV128


def _prep_body(x_ref, spread, ones_b, g1_ref):
    i = pl.program_id(0)
    g1 = _dot(x_ref[...], spread[...]) + ones_b[...]
    g1_ref[...] = jnp.where(_valid_rows(i), g1, 0.0)


def _dense1_body(p_ref, x_ref, mdeg, kwl1, kwr1, b1p, kwl2a, kwl2b, kwr2a,
                 kwr2b, g2a_ref, g2b_ref, r2a_ref, r2b_ref, dinv_ref):
    i = pl.program_id(0)
    p = p_ref[...]
    agg = p[0] + p[1]
    deg_b = _dot(agg, mdeg[...])
    dinv = 1.0 / jnp.maximum(deg_b, 1.0)
    mean = agg * dinv
    h1 = jnp.maximum(_dot(mean, kwl1[...]) + _dot(x_ref[...], kwr1[...])
                     + b1p[...], 0.0)
    valid = _valid_rows(i)
    g2a_ref[...] = jnp.where(valid, _dot(h1, kwl2a[...]), 0.0)
    g2b_ref[...] = jnp.where(valid, _dot(h1, kwl2b[...]), 0.0)
    r2a_ref[...] = _dot(h1, kwr2a[...])
    r2b_ref[...] = _dot(h1, kwr2b[...])
    dinv_ref[...] = dinv


def _dense2_body(p2a_ref, p2b_ref, r2a_ref, r2b_ref, dinv_ref, b2ap, b2bp,
                 kwl3a, kwl3b, kwr3a, kwr3b, g3_ref, r3_ref):
    i = pl.program_id(0)
    p2a = p2a_ref[...]
    p2b = p2b_ref[...]
    dinv = dinv_ref[...]
    h2a = jnp.maximum((p2a[0] + p2a[1]) * dinv + b2ap[...] + r2a_ref[...], 0.0)
    h2b = jnp.maximum((p2b[0] + p2b[1]) * dinv + b2bp[...] + r2b_ref[...], 0.0)
    g3 = _dot(h2a, kwl3a[...]) + _dot(h2b, kwl3b[...])
    g3_ref[...] = jnp.where(_valid_rows(i), g3, 0.0)
    r3_ref[...] = _dot(h2a, kwr3a[...]) + _dot(h2b, kwr3b[...])


def _dense3_body(p3_ref, r3_ref, dinv_ref, b3p, kfc1, f1bp, kfc2, f2bp,
                 sel0, sel1, cout, out_ref):
    p3 = p3_ref[...]
    h3 = jnp.maximum((p3[0] + p3[1]) * dinv_ref[...] + b3p[...] + r3_ref[...],
                     0.0)
    f1 = jnp.maximum(_dot(h3, kfc1[...]) + f1bp[...], 0.0)
    lg = _dot(f1, kfc2[...]) + f2bp[...]
    l0 = _dot(lg, sel0[...])
    l1 = _dot(lg, sel1[...])
    m = jnp.maximum(l0, l1)
    lse = m + jnp.log(jnp.exp(l0 - m) + jnp.exp(l1 - m))
    out_ref[...] = _dot(lg - lse, cout[...])


def _blk(width):
    return pl.BlockSpec((BP, width), lambda i: (i, 0))


def _p_spec():
    return pl.BlockSpec((2, BP, 128), lambda i: (0, i, 0))


def _w_spec(shape):
    return pl.BlockSpec(shape, lambda i: tuple(0 for _ in shape))


def _shape(width):
    return jax.ShapeDtypeStruct((NP128, width), jnp.float32)


def kernel(x, edge_index, Wl1, bl1, Wr1, Wl2, bl2, Wr2, Wl3, bl3, Wr3,
           fc1_w, fc1_b, fc2_w, fc2_b):
    n = x.shape[0]
    e = edge_index.shape[1]
    assert n == N_NODES and e == N_EDGES
    f32 = jnp.float32

    edges2d = jnp.pad(edge_index, ((0, 0), (0, E_PAD - e)),
                      constant_values=n).reshape(2, IDX_ROWS, 128)

    x64 = jnp.pad(x.reshape(NV128, 64), ((0, NP128 - NV128), (0, 0)))

    i8 = jnp.eye(8, dtype=f32)
    kwr1 = jnp.kron(i8, Wr1.T)
    kwl1 = jnp.kron(i8, jnp.pad(Wl1.T, ((0, 8), (0, 0))))
    kwl2a = jnp.kron(i8, Wl2.T[:, 0:16])
    kwl2b = jnp.kron(i8, Wl2.T[:, 16:32])
    kwr2a = jnp.kron(i8, Wr2.T[:, 0:16])
    kwr2b = jnp.kron(i8, Wr2.T[:, 16:32])
    kwl3a = jnp.kron(i8, Wl3.T[0:16])
    kwl3b = jnp.kron(i8, Wl3.T[16:32])
    kwr3a = jnp.kron(i8, Wr3.T[0:16])
    kwr3b = jnp.kron(i8, Wr3.T[16:32])
    kfc1 = jnp.kron(i8, jnp.pad(fc1_w.T, ((0, 0), (0, 8))))
    kfc2 = jnp.kron(i8, jnp.pad(fc2_w.T, ((0, 8), (0, 14))))
    b1p = jnp.tile(bl1, 8).reshape(1, 512)
    b2ap = jnp.tile(bl2[0:16], 8).reshape(1, 128)
    b2bp = jnp.tile(bl2[16:32], 8).reshape(1, 128)
    b3p = jnp.tile(bl3, 8).reshape(1, 128)
    f1bp = jnp.tile(jnp.pad(fc1_b, (0, 8)), 8).reshape(1, 128)
    f2bp = jnp.tile(jnp.pad(fc2_b, (0, 14)), 8).reshape(1, 128)

    grid = (NP128 // BP,)

    g1 = pl.pallas_call(
        _prep_body, grid=grid,
        in_specs=[_blk(64), _w_spec((64, 128)), _w_spec((1, 128))],
        out_specs=[_blk(128)],
        out_shape=[_shape(128)],
    )(x64, jnp.asarray(_SPREAD), jnp.asarray(_ONES_B))[0]
    p1 = _sc_segment_sum(g1, edges2d)

    g2a, g2b, r2a, r2b, dinv = pl.pallas_call(
        _dense1_body, grid=grid,
        in_specs=[_p_spec(), _blk(64), _w_spec((128, 128)),
                  _w_spec((128, 512)), _w_spec((64, 512)),
                  _w_spec((1, 512))] + [_w_spec((512, 128))] * 4,
        out_specs=[_blk(128)] * 5,
        out_shape=[_shape(128)] * 5,
    )(p1, x64, jnp.asarray(_M_DEG), kwl1, kwr1, b1p, kwl2a, kwl2b, kwr2a,
      kwr2b)

    p2a = _sc_segment_sum(g2a, edges2d)
    p2b = _sc_segment_sum(g2b, edges2d)

    g3, r3 = pl.pallas_call(
        _dense2_body, grid=grid,
        in_specs=[_p_spec(), _p_spec(), _blk(128), _blk(128), _blk(128),
                  _w_spec((1, 128)), _w_spec((1, 128)),
                  _w_spec((128, 128)), _w_spec((128, 128)),
                  _w_spec((128, 128)), _w_spec((128, 128))],
        out_specs=[_blk(128)] * 2,
        out_shape=[_shape(128)] * 2,
    )(p2a, p2b, r2a, r2b, dinv, b2ap, b2bp, kwl3a, kwl3b, kwr3a, kwr3b)

    p3 = _sc_segment_sum(g3, edges2d)

    out16 = pl.pallas_call(
        _dense3_body, grid=grid,
        in_specs=[_p_spec(), _blk(128), _blk(128), _w_spec((1, 128)),
                  _w_spec((128, 128)), _w_spec((1, 128)),
                  _w_spec((128, 128)), _w_spec((1, 128)),
                  _w_spec((128, 128)), _w_spec((128, 128)),
                  _w_spec((128, 16))],
        out_specs=[_blk(16)],
        out_shape=[_shape(16)],
    )(p3, r3, dinv, b3p, kfc1, f1bp, kfc2, f2bp,
      jnp.asarray(_SEL0), jnp.asarray(_SEL1), jnp.asarray(_C_OUT))[0]

    return out16[:NV128].reshape(N_NODES, 2)
